# Initial kernel scaffold; baseline (speedup 1.0000x reference)
import jax
import jax.numpy as jnp
from jax.experimental import pallas as pl


def kernel(x, edge_index, edge_attr, W1, b1, W2, b2, W3, b3, pw, Wh, bh, g1, be1, Wd, bd, g2, be2, Wdc, bdc):
    raise NotImplementedError("write your pallas kernel here")


# speedup vs baseline: 1.1107x; 1.1107x over previous
import jax
import jax.numpy as jnp
from jax.experimental import pallas as pl

N = 256
V = 68
D = 1
EPG = 544
E = N * EPG
C1, C2, C3 = 512, 256, 128
K = 34
SEQ = C3 * K
LAT = 128
KMAX = 40

_f32 = jnp.float32
_HI = jax.lax.Precision.HIGHEST


def _dot_hi(a, b):
    return jax.lax.dot_general(a, b, (((1,), (0,)), ((), ())), precision=_HI)


def _gcn_kernel(colsS_ref, colsL_ref, rowsL_ref, ewL_ref, xS_ref, W1_ref,
                b1_ref, W2_ref, b2_ref, W3_ref, b3_ref, pw_ref, Wh_ref,
                bh_ref, g1_ref, be1_ref, Wd_ref, bd_ref, g2_ref, be2_ref,
                Wdc_ref, bdc_ref, out_ref):
    colsS = colsS_ref[...].reshape(EPG, 1)
    colsL = colsL_ref[...].reshape(1, EPG)
    rowsL = rowsL_ref[...].reshape(1, EPG)
    ewL = ewL_ref[...].reshape(1, EPG)
    xS = xS_ref[...].reshape(V, 1)

    lane_v = jax.lax.broadcasted_iota(jnp.int32, (1, V), 1)
    sub_v = jax.lax.broadcasted_iota(jnp.int32, (V, 1), 0)
    Cm = (colsS == lane_v).astype(_f32)
    CTf = (sub_v == colsL).astype(_f32)
    RT = (sub_v == rowsL).astype(_f32)

    bstart_row = jnp.sum((colsS < lane_v).astype(_f32), axis=0,
                         keepdims=True)
    e_iota = jax.lax.broadcasted_iota(jnp.int32, (1, EPG), 1).astype(_f32)
    bstartE = jnp.sum(CTf * jnp.swapaxes(bstart_row, 0, 1),
                      axis=0, keepdims=True)
    rL = (e_iota - bstartE).astype(jnp.int32)
    k_sub = jax.lax.broadcasted_iota(jnp.int32, (KMAX, 1), 0)
    A = (k_sub == rL).astype(_f32)

    def slot(valsL):
        rows_ = []
        for k in range(KMAX):
            m = (A[k:k + 1, :] * valsL)
            rows_.append(jnp.sum(Cm * jnp.swapaxes(m, 0, 1),
                                 axis=0, keepdims=True))
        return jnp.concatenate(rows_, axis=0)

    slot_ew = slot(ewL)

    deg = jnp.zeros((1, V), _f32)
    for k in range(KMAX):
        deg = deg + slot_ew[k:k + 1, :]
    deg = deg + 1.0
    disR = jnp.where(deg > 0, jax.lax.rsqrt(jnp.maximum(deg, 1e-12)), 0.0)
    disC = jnp.swapaxes(disR, 0, 1)
    dis_rL = jnp.sum(RT * disC, axis=0, keepdims=True)
    dis_cL = jnp.sum(CTf * disC, axis=0, keepdims=True)
    normL = (dis_rL * ewL) * dis_cL
    disq = disC * disC

    slot_norm = slot(normL)
    slot_src = slot(rowsL.astype(_f32))

    Gks, normks = [], []
    for k in range(KMAX):
        srck = jnp.swapaxes(slot_src[k:k + 1, :], 0, 1).astype(jnp.int32)
        Gks.append((srck == lane_v).astype(_f32))
        normks.append(jnp.swapaxes(slot_norm[k:k + 1, :], 0, 1))

    def agg(xw):
        out = jnp.zeros(xw.shape, _f32)
        for k in range(KMAX):
            out = out + _dot_hi(Gks[k], xw) * normks[k]
        return out

    xw1 = xS * W1_ref[...]
    h1 = jnp.maximum(agg(xw1) + xw1 * disq + b1_ref[...], 0.0)

    xw2 = h1 @ W2_ref[...]
    h2 = jnp.maximum(agg(xw2) + xw2 * disq + b2_ref[...], 0.0)

    xw3 = h2 @ W3_ref[...]
    h3 = jnp.maximum(agg(xw3) + xw3 * disq + b3_ref[...], 0.0)

    pwc = pw_ref[...]
    pn = jnp.sqrt(jnp.sum(pwc * pwc))
    score = jnp.tanh((h3 @ pwc) / pn)
    scoreT = score.reshape(1, V)

    vv_j = jax.lax.broadcasted_iota(jnp.int32, (V, V), 1)
    vv_i = jax.lax.broadcasted_iota(jnp.int32, (V, V), 0)
    before = (scoreT > score) | ((scoreT == score) & (vv_j < vv_i))
    rank = jnp.sum(before.astype(jnp.int32), axis=1, keepdims=True)
    k_iota = jax.lax.broadcasted_iota(jnp.int32, (K, V), 0)
    P = (rank.reshape(1, V) == k_iota).astype(_f32)
    vals = _dot_hi(P, score)
    sel = _dot_hi(P, h3)
    z3 = sel * vals

    zg = z3.reshape(1, K * C3)
    zh = zg @ Wh_ref[...] + bh_ref[...]
    mu = jnp.mean(zh, axis=1, keepdims=True)
    var = jnp.mean((zh - mu) * (zh - mu), axis=1, keepdims=True)
    t = (zh - mu) * jax.lax.rsqrt(var + 1e-5) * g1_ref[...] + be1_ref[...]

    d = jnp.maximum(t @ Wd_ref[...] + bd_ref[...], 0.0)
    mu2 = jnp.mean(d, axis=1, keepdims=True)
    var2 = jnp.mean((d - mu2) * (d - mu2), axis=1, keepdims=True)
    t2 = (d - mu2) * jax.lax.rsqrt(var2 + 1e-5) * g2_ref[...] + be2_ref[...]

    xwq = jnp.swapaxes(t2, 0, 1) * Wdc_ref[0, 0]
    outg = jnp.maximum(agg(xwq) + xwq * disq + bdc_ref[0, 0], 0.0)
    out_ref[...] = outg.reshape(1, 1, V)


def kernel(x, edge_index, edge_attr, W1, b1, W2, b2, W3, b3, pw, Wh, bh,
           g1, be1, Wd, bd, g2, be2, Wdc, bdc):
    rows2 = edge_index[0].reshape(N, EPG)
    cols2 = edge_index[1].reshape(N, EPG)
    ew2 = edge_attr.reshape(N, EPG)
    order = jnp.argsort(cols2, axis=1, stable=True)
    rows2 = jnp.take_along_axis(rows2, order, axis=1)
    ew2 = jnp.take_along_axis(ew2, order, axis=1)
    cols2 = jnp.take_along_axis(cols2, order, axis=1)

    colsS = cols2.reshape(N, EPG, 1)
    colsL = cols2.reshape(N, 1, EPG)
    rowsL = rows2.reshape(N, 1, EPG)
    ewL = ew2.reshape(N, 1, EPG)
    xS = x.reshape(N, V, 1)

    def full2(shape):
        return pl.BlockSpec(shape, lambda i: (0, 0))

    out = pl.pallas_call(
        _gcn_kernel,
        grid=(N,),
        in_specs=[
            pl.BlockSpec((1, EPG, 1), lambda i: (i, 0, 0)),
            pl.BlockSpec((1, 1, EPG), lambda i: (i, 0, 0)),
            pl.BlockSpec((1, 1, EPG), lambda i: (i, 0, 0)),
            pl.BlockSpec((1, 1, EPG), lambda i: (i, 0, 0)),
            pl.BlockSpec((1, V, 1), lambda i: (i, 0, 0)),
            full2((1, C1)),
            full2((1, C1)),
            full2((C1, C2)),
            full2((1, C2)),
            full2((C2, C3)),
            full2((1, C3)),
            full2((C3, 1)),
            full2((SEQ, LAT)),
            full2((1, LAT)),
            full2((1, LAT)),
            full2((1, LAT)),
            full2((LAT, V)),
            full2((1, V)),
            full2((1, V)),
            full2((1, V)),
            full2((1, 1)),
            full2((1, 1)),
        ],
        out_specs=pl.BlockSpec((1, 1, V), lambda i: (i, 0, 0)),
        out_shape=jax.ShapeDtypeStruct((N, 1, V), jnp.float32),
    )(colsS, colsL, rowsL, ewL, xS, W1, b1.reshape(1, -1), W2,
      b2.reshape(1, -1), W3, b3.reshape(1, -1), pw.reshape(-1, 1), Wh,
      bh.reshape(1, -1), g1.reshape(1, -1), be1.reshape(1, -1), Wd,
      bd.reshape(1, -1), g2.reshape(1, -1), be2.reshape(1, -1),
      Wdc, bdc.reshape(1, 1))
    return out.reshape(N, V, D)
